# Initial kernel scaffold; baseline (speedup 1.0000x reference)
import jax
import jax.numpy as jnp
from jax.experimental import pallas as pl


def kernel(classifications, regressions, anchors, annotations):
    raise NotImplementedError("write your pallas kernel here")


# speedup vs baseline: 33.7674x; 33.7674x over previous
import functools
import math

import jax
import jax.numpy as jnp
from jax.experimental import pallas as pl
from jax.experimental.pallas import tpu as pltpu

_ALPHA = 0.25
_HOR_THR = 0.4
_ROT_THR = 0.2
_D2R = math.pi / 180.0
_A_BLK = 1024


def _fl_kernel(a_total, cls_ref, reg_ref, anc_ref, ann_ref, out_ref):
    blk = pl.program_id(0)
    b = pl.program_id(1)
    C = cls_ref.shape[2]
    M = ann_ref.shape[1]
    f32 = jnp.float32

    lane = jax.lax.broadcasted_iota(jnp.int32, (1, _A_BLK), 1)
    valid = (blk * _A_BLK + lane) < a_total

    cls_t = jnp.where(valid, cls_ref[0].T, 0.5)
    reg_t = jnp.where(valid, reg_ref[0].T, 0.0)
    anc_t = jnp.where(valid, anc_ref[0].T, 0.0)
    annT = ann_ref[0].T

    g_cx, g_cy = annT[0:1], annT[1:2]
    g_w, g_h = annT[2:3], annT[3:4]
    g_th, g_cls = annT[4:5], annT[5:6]
    ang = g_th * _D2R
    co = jnp.cos(ang) * 0.5
    si = jnp.sin(ang) * 0.5
    p0x = g_cx - si * g_h - co * g_w
    p0y = g_cy + co * g_h - si * g_w
    p1x = g_cx + si * g_h - co * g_w
    p1y = g_cy - co * g_h - si * g_w
    p2x = 2.0 * g_cx - p0x
    p2y = 2.0 * g_cy - p0y
    p3x = 2.0 * g_cx - p1x
    p3y = 2.0 * g_cy - p1y
    bx1 = jnp.minimum(jnp.minimum(p0x, p1x), jnp.minimum(p2x, p3x))
    by1 = jnp.minimum(jnp.minimum(p0y, p1y), jnp.minimum(p2y, p3y))
    bx2 = jnp.maximum(jnp.maximum(p0x, p1x), jnp.maximum(p2x, p3x))
    by2 = jnp.maximum(jnp.maximum(p0y, p1y), jnp.maximum(p2y, p3y))
    area_b = (bx2 - bx1) * (by2 - by1)
    qarea = 0.5 * jnp.abs(
        p0x * p1y - p1x * p0y + p1x * p2y - p2x * p1y
        + p2x * p3y - p3x * p2y + p3x * p0y - p0x * p3y)

    rect = jnp.concatenate(
        [bx1, by1, bx2, by2, area_b, area_b, area_b, area_b], axis=0)
    rectT = rect.T
    cbx1, cby1 = rectT[:, 0:1], rectT[:, 1:2]
    cbx2, cby2 = rectT[:, 2:3], rectT[:, 3:4]
    carea_b = rectT[:, 4:5]

    ax1, ay1 = anc_t[0:1], anc_t[1:2]
    ax2, ay2 = anc_t[2:3], anc_t[3:4]
    ath = anc_t[4:5]
    area_a = (ax2 - ax1) * (ay2 - ay1)

    iw = jnp.maximum(jnp.minimum(ax2, cbx2) - jnp.maximum(ax1, cbx1), 0.0)
    ih = jnp.maximum(jnp.minimum(ay2, cby2) - jnp.maximum(ay1, cby1), 0.0)
    inter_h = iw * ih
    ua = jnp.maximum(area_a + carea_b - inter_h, 1e-8)
    hiou = inter_h / ua
    iou_max = jnp.max(hiou, axis=0, keepdims=True)
    midx = jax.lax.broadcasted_iota(jnp.int32, (M, _A_BLK), 0)
    am = jnp.min(jnp.where(hiou == iou_max, midx, M), axis=0, keepdims=True)
    oh = (midx == am).astype(f32)
    hor_pos = (iou_max >= _HOR_THR) & valid

    vals16 = jnp.concatenate(
        [g_cx, g_cy, g_w, g_h, g_th, g_cls,
         p0x, p0y, p1x, p1y, p2x, p2y, p3x, p3y, qarea, qarea], axis=0)
    gath = jnp.dot(vals16, oh, preferred_element_type=f32)
    s_cx, s_cy = gath[0:1], gath[1:2]
    s_w, s_h = gath[2:3], gath[3:4]
    s_th, s_cls = gath[4:5], gath[5:6]
    s_qa = gath[14:15]

    zero4 = jnp.zeros((4, _A_BLK), f32)
    px = jnp.concatenate([gath[6:7], gath[8:9], gath[10:11], gath[12:13], zero4], 0)
    py = jnp.concatenate([gath[7:8], gath[9:10], gath[11:12], gath[13:14], zero4], 0)
    n = jnp.full((1, _A_BLK), 4, jnp.int32)
    idx8 = jax.lax.broadcasted_iota(jnp.int32, (8, _A_BLK), 0)

    for use_x, bound, sign in ((True, ax1, 1.0), (True, ax2, -1.0),
                               (False, ay1, 1.0), (False, ay2, -1.0)):
        coord = px if use_x else py
        dp = sign * (coord - bound)
        is_last = idx8 == (n - 1)
        valid_v = idx8 < n
        qx = jnp.where(is_last, px[0:1], jnp.roll(px, -1, axis=0))
        qy = jnp.where(is_last, py[0:1], jnp.roll(py, -1, axis=0))
        dq = jnp.where(is_last, dp[0:1], jnp.roll(dp, -1, axis=0))
        in_p = dp >= 0.0
        in_q = dq >= 0.0
        denom = dp - dq
        t = dp / jnp.where(jnp.abs(denom) < 1e-9, 1e-9, denom)
        ix = px + t * (qx - px)
        iy = py + t * (qy - py)
        k0 = in_p & valid_v
        k1 = (in_p ^ in_q) & valid_v
        c0 = k0.astype(jnp.int32)
        c01 = c0 + k1.astype(jnp.int32)
        run = c01
        for s in (1, 2, 4):
            run = run + jnp.where(idx8 >= s, jnp.roll(run, s, axis=0), 0)
        excl = run - c01
        posP = excl
        posI = excl + c0
        rows_x = []
        rows_y = []
        for j in range(8):
            m0 = k0 & (posP == j)
            m1 = k1 & (posI == j)
            rows_x.append(jnp.sum(jnp.where(m0, px, 0.0) + jnp.where(m1, ix, 0.0),
                                  axis=0, keepdims=True))
            rows_y.append(jnp.sum(jnp.where(m0, py, 0.0) + jnp.where(m1, iy, 0.0),
                                  axis=0, keepdims=True))
        px = jnp.concatenate(rows_x, axis=0)
        py = jnp.concatenate(rows_y, axis=0)
        n = jnp.minimum(jnp.sum(c01, axis=0, keepdims=True), 8)

    valid_v = idx8 < n
    is_last = idx8 == (n - 1)
    qx = jnp.where(is_last, px[0:1], jnp.roll(px, -1, axis=0))
    qy = jnp.where(is_last, py[0:1], jnp.roll(py, -1, axis=0))
    cross = px * qy - qx * py
    inter_s = 0.5 * jnp.abs(jnp.sum(jnp.where(valid_v, cross, 0.0),
                                    axis=0, keepdims=True))
    union = jnp.maximum(s_qa + area_a - inter_s, 1e-8)
    siou = inter_s / union
    pos = hor_pos & (siou >= _ROT_THR)
    npos = jnp.sum(jnp.where(pos, 1.0, 0.0))

    clsv = jnp.clip(cls_t, 1e-4, 1.0 - 1e-4)
    cio = jax.lax.broadcasted_iota(jnp.int32, (C, _A_BLK), 0)
    is_one = pos & (cio == s_cls.astype(jnp.int32))
    af = jnp.where(is_one, _ALPHA, 1.0 - _ALPHA)
    ptv = jnp.where(is_one, 1.0 - clsv, clsv)
    bce = -jnp.log(jnp.where(is_one, clsv, 1.0 - clsv))
    cls_sum = jnp.sum(jnp.where(hor_pos, af * ptv * ptv * bce, 0.0))

    aw = ax2 - ax1
    ah = ay2 - ay1
    acx = ax1 + 0.5 * aw
    acy = ay1 + 0.5 * ah
    tdx = (s_cx - acx) / aw
    tdy = (s_cy - acy) / ah
    tdw = jnp.log(jnp.maximum(s_w, 1.0) / aw)
    tdh = jnp.log(jnp.maximum(s_h, 1.0) / ah)
    tth = (s_th - ath) * _D2R
    rt = jnp.concatenate([tdy, tdx, tdh, tdw, tth], axis=0)
    diff = jnp.abs(rt - reg_t[0:5])
    sl1 = jnp.where(diff <= 1.0 / 9.0, 4.5 * diff * diff, diff - 0.5 / 9.0)
    reg_sum = jnp.sum(jnp.where(pos, sl1, 0.0))

    r8 = jax.lax.broadcasted_iota(jnp.int32, (8, 128), 0)
    l8 = jax.lax.broadcasted_iota(jnp.int32, (8, 128), 1)
    row_b = r8 == b
    contrib = (jnp.where(row_b & (l8 == 0), cls_sum, 0.0)
               + jnp.where(row_b & (l8 == 1), reg_sum, 0.0)
               + jnp.where(row_b & (l8 == 2), npos, 0.0))

    @pl.when(b == 0)
    def _():
        out_ref[...] = jnp.zeros_like(out_ref)

    out_ref[...] += contrib[None]


@jax.jit
def _run(classifications, regressions, anchors, annotations):
    B, A, C = classifications.shape
    M = annotations.shape[1]
    nblk = pl.cdiv(A, _A_BLK)
    reg_p = jnp.pad(regressions, ((0, 0), (0, 0), (0, 3)))
    anc_p = jnp.pad(anchors, ((0, 0), (0, 0), (0, 3)))
    ann_p = jnp.pad(annotations, ((0, 0), (0, 0), (0, 2)))
    part = pl.pallas_call(
        functools.partial(_fl_kernel, A),
        out_shape=jax.ShapeDtypeStruct((nblk, 8, 128), jnp.float32),
        grid=(nblk, B),
        in_specs=[
            pl.BlockSpec((1, _A_BLK, C), lambda i, j: (j, i, 0)),
            pl.BlockSpec((1, _A_BLK, 8), lambda i, j: (j, i, 0)),
            pl.BlockSpec((1, _A_BLK, 8), lambda i, j: (0, i, 0)),
            pl.BlockSpec((1, M, 8), lambda i, j: (j, 0, 0)),
        ],
        out_specs=pl.BlockSpec((1, 8, 128), lambda i, j: (i, 0, 0)),
        compiler_params=pltpu.CompilerParams(
            dimension_semantics=("parallel", "arbitrary"),
        ),
        name="rot_focal_loss",
    )(classifications, reg_p, anc_p, ann_p)
    s = part.sum(0)
    cls_s, reg_s, npv = s[:B, 0], s[:B, 1], s[:B, 2]
    cls_l = cls_s / jnp.maximum(npv, 1.0)
    reg_l = reg_s / jnp.maximum(npv * 5.0, 1.0)
    return jnp.stack([cls_l.mean(), reg_l.mean()])


def kernel(classifications, regressions, anchors, annotations):
    return _run(classifications, regressions, anchors, annotations)


# speedup vs baseline: 36.1944x; 1.0719x over previous
import functools
import math

import jax
import jax.numpy as jnp
from jax.experimental import pallas as pl
from jax.experimental.pallas import tpu as pltpu

_ALPHA = 0.25
_HOR_THR = 0.4
_ROT_THR = 0.2
_D2R = math.pi / 180.0
_A_BLK = 2048


def _fl_kernel(a_total, cls_ref, reg_ref, anc_ref, ann_ref, out_ref,
               gt_ref, rect_ref):
    b = pl.program_id(0)
    blk = pl.program_id(1)
    C = cls_ref.shape[2]
    M = ann_ref.shape[1]
    f32 = jnp.float32

    @pl.when(blk == 0)
    def _():
        annT = ann_ref[0].T
        g_cx, g_cy = annT[0:1], annT[1:2]
        g_w, g_h = annT[2:3], annT[3:4]
        g_th, g_cls = annT[4:5], annT[5:6]
        ang = g_th * _D2R
        co = jnp.cos(ang) * 0.5
        si = jnp.sin(ang) * 0.5
        p0x = g_cx - si * g_h - co * g_w
        p0y = g_cy + co * g_h - si * g_w
        p1x = g_cx + si * g_h - co * g_w
        p1y = g_cy - co * g_h - si * g_w
        p2x = 2.0 * g_cx - p0x
        p2y = 2.0 * g_cy - p0y
        p3x = 2.0 * g_cx - p1x
        p3y = 2.0 * g_cy - p1y
        bx1 = jnp.minimum(jnp.minimum(p0x, p1x), jnp.minimum(p2x, p3x))
        by1 = jnp.minimum(jnp.minimum(p0y, p1y), jnp.minimum(p2y, p3y))
        bx2 = jnp.maximum(jnp.maximum(p0x, p1x), jnp.maximum(p2x, p3x))
        by2 = jnp.maximum(jnp.maximum(p0y, p1y), jnp.maximum(p2y, p3y))
        area_b = (bx2 - bx1) * (by2 - by1)
        qarea = 0.5 * jnp.abs(
            p0x * p1y - p1x * p0y + p1x * p2y - p2x * p1y
            + p2x * p3y - p3x * p2y + p3x * p0y - p0x * p3y)
        gt_ref[...] = jnp.concatenate(
            [g_cx, g_cy, g_w, g_h, g_th, g_cls,
             p0x, p0y, p1x, p1y, p2x, p2y, p3x, p3y, qarea, qarea], axis=0)
        rect = jnp.concatenate(
            [bx1, by1, bx2, by2, area_b, area_b, area_b, area_b], axis=0)
        rect_ref[...] = rect.T

    lane = jax.lax.broadcasted_iota(jnp.int32, (1, _A_BLK), 1)
    valid = (blk * _A_BLK + lane) < a_total

    cls_t = jnp.where(valid, cls_ref[0].T, 0.5)
    reg_t = jnp.where(valid, reg_ref[0].T, 0.0)
    anc_t = jnp.where(valid, anc_ref[0].T, 0.0)

    vals16 = gt_ref[...]
    cbx1, cby1 = rect_ref[:, 0:1], rect_ref[:, 1:2]
    cbx2, cby2 = rect_ref[:, 2:3], rect_ref[:, 3:4]
    carea_b = rect_ref[:, 4:5]

    ax1, ay1 = anc_t[0:1], anc_t[1:2]
    ax2, ay2 = anc_t[2:3], anc_t[3:4]
    ath = anc_t[4:5]
    area_a = (ax2 - ax1) * (ay2 - ay1)

    iw = jnp.maximum(jnp.minimum(ax2, cbx2) - jnp.maximum(ax1, cbx1), 0.0)
    ih = jnp.maximum(jnp.minimum(ay2, cby2) - jnp.maximum(ay1, cby1), 0.0)
    inter_h = iw * ih
    ua = jnp.maximum(area_a + carea_b - inter_h, 1e-8)
    hiou = inter_h / ua
    iou_max = jnp.max(hiou, axis=0, keepdims=True)
    midx = jax.lax.broadcasted_iota(jnp.int32, (M, _A_BLK), 0)
    am = jnp.min(jnp.where(hiou == iou_max, midx, M), axis=0, keepdims=True)
    oh = (midx == am).astype(f32)
    hor_pos = (iou_max >= _HOR_THR) & valid

    gath = jnp.dot(vals16, oh, preferred_element_type=f32)
    s_cx, s_cy = gath[0:1], gath[1:2]
    s_w, s_h = gath[2:3], gath[3:4]
    s_th, s_cls = gath[4:5], gath[5:6]
    s_qa = gath[14:15]

    zero4 = jnp.zeros((4, _A_BLK), f32)
    px = jnp.concatenate([gath[6:7], gath[8:9], gath[10:11], gath[12:13]], 0)
    py = jnp.concatenate([gath[7:8], gath[9:10], gath[11:12], gath[13:14]], 0)
    px = jnp.concatenate([px, zero4], 0)
    py = jnp.concatenate([py, zero4], 0)
    n = jnp.full((1, _A_BLK), 4, jnp.int32)
    idx8 = jax.lax.broadcasted_iota(jnp.int32, (8, _A_BLK), 0)

    for use_x, bound, sign, jmax in ((True, ax1, 1.0, 5),
                                     (True, ax2, -1.0, 6),
                                     (False, ay1, 1.0, 7)):
        coord = px if use_x else py
        dp = sign * (coord - bound)
        is_last = idx8 == (n - 1)
        valid_v = idx8 < n
        qx = jnp.where(is_last, px[0:1], jnp.roll(px, -1, axis=0))
        qy = jnp.where(is_last, py[0:1], jnp.roll(py, -1, axis=0))
        dq = jnp.where(is_last, dp[0:1], jnp.roll(dp, -1, axis=0))
        in_p = dp >= 0.0
        in_q = dq >= 0.0
        denom = dp - dq
        t = dp / jnp.where(jnp.abs(denom) < 1e-9, 1e-9, denom)
        ix = px + t * (qx - px)
        iy = py + t * (qy - py)
        k0 = in_p & valid_v
        k1 = (in_p ^ in_q) & valid_v
        c0 = k0.astype(jnp.int32)
        c01 = c0 + k1.astype(jnp.int32)
        run = c01
        for s in (1, 2, 4):
            run = run + jnp.where(idx8 >= s, jnp.roll(run, s, axis=0), 0)
        posP = run - c01
        posI = posP + c0
        rows_x = []
        rows_y = []
        for j in range(jmax):
            m0 = k0 & (posP == j)
            m1 = k1 & (posI == j)
            rows_x.append(jnp.sum(jnp.where(m0, px, 0.0) + jnp.where(m1, ix, 0.0),
                                  axis=0, keepdims=True))
            rows_y.append(jnp.sum(jnp.where(m0, py, 0.0) + jnp.where(m1, iy, 0.0),
                                  axis=0, keepdims=True))
        pad = jnp.zeros((8 - jmax, _A_BLK), f32)
        px = jnp.concatenate(rows_x + [pad], axis=0)
        py = jnp.concatenate(rows_y + [pad], axis=0)
        n = jnp.minimum(jnp.sum(c01, axis=0, keepdims=True), jmax)

    dp = ay2 - py
    is_last = idx8 == (n - 1)
    valid_v = idx8 < n
    qx = jnp.where(is_last, px[0:1], jnp.roll(px, -1, axis=0))
    qy = jnp.where(is_last, py[0:1], jnp.roll(py, -1, axis=0))
    dq = jnp.where(is_last, dp[0:1], jnp.roll(dp, -1, axis=0))
    in_p = dp >= 0.0
    in_q = dq >= 0.0
    denom = dp - dq
    t = dp / jnp.where(jnp.abs(denom) < 1e-9, 1e-9, denom)
    ix = px + t * (qx - px)
    iy = py + t * (qy - py)
    k0 = in_p & valid_v
    k1 = (in_p ^ in_q) & valid_v
    idx16 = jax.lax.broadcasted_iota(jnp.int32, (16, _A_BLK), 0)
    even = (idx16 % 2) == 0
    candx = jnp.where(even, jnp.repeat(px, 2, axis=0), jnp.repeat(ix, 2, axis=0))
    candy = jnp.where(even, jnp.repeat(py, 2, axis=0), jnp.repeat(iy, 2, axis=0))
    kf = jnp.where(even, jnp.repeat(k0.astype(f32), 2, axis=0),
                   jnp.repeat(k1.astype(f32), 2, axis=0))
    wx = jnp.roll(candx, -1, axis=0)
    wy = jnp.roll(candy, -1, axis=0)
    have = jnp.roll(kf, -1, axis=0)
    for s in (1, 2, 4, 8):
        hmask = have > 0.5
        wx = jnp.where(hmask, wx, jnp.roll(wx, -s, axis=0))
        wy = jnp.where(hmask, wy, jnp.roll(wy, -s, axis=0))
        have = jnp.maximum(have, jnp.roll(have, -s, axis=0))
    cross = candx * wy - wx * candy
    inter_s = 0.5 * jnp.abs(jnp.sum(jnp.where(kf > 0.5, cross, 0.0),
                                    axis=0, keepdims=True))
    union = jnp.maximum(s_qa + area_a - inter_s, 1e-8)
    siou = inter_s / union
    pos = hor_pos & (siou >= _ROT_THR)
    npos = jnp.sum(jnp.where(pos, 1.0, 0.0))

    clsv = jnp.clip(cls_t, 1e-4, 1.0 - 1e-4)
    cio = jax.lax.broadcasted_iota(jnp.int32, (C, _A_BLK), 0)
    is_one = pos & (cio == s_cls.astype(jnp.int32))
    af = jnp.where(is_one, _ALPHA, 1.0 - _ALPHA)
    ptv = jnp.where(is_one, 1.0 - clsv, clsv)
    bce = -jnp.log(jnp.where(is_one, clsv, 1.0 - clsv))
    cls_sum = jnp.sum(jnp.where(hor_pos, af * ptv * ptv * bce, 0.0))

    aw = ax2 - ax1
    ah = ay2 - ay1
    acx = ax1 + 0.5 * aw
    acy = ay1 + 0.5 * ah
    tdx = (s_cx - acx) / aw
    tdy = (s_cy - acy) / ah
    tdw = jnp.log(jnp.maximum(s_w, 1.0) / aw)
    tdh = jnp.log(jnp.maximum(s_h, 1.0) / ah)
    tth = (s_th - ath) * _D2R
    rt = jnp.concatenate([tdy, tdx, tdh, tdw, tth], axis=0)
    diff = jnp.abs(rt - reg_t[0:5])
    sl1 = jnp.where(diff <= 1.0 / 9.0, 4.5 * diff * diff, diff - 0.5 / 9.0)
    reg_sum = jnp.sum(jnp.where(pos, sl1, 0.0))

    l1 = jax.lax.broadcasted_iota(jnp.int32, (1, 128), 1)
    contrib = (jnp.where(l1 == 0, cls_sum, 0.0)
               + jnp.where(l1 == 1, reg_sum, 0.0)
               + jnp.where(l1 == 2, npos, 0.0))

    @pl.when(blk == 0)
    def _():
        out_ref[...] = jnp.zeros_like(out_ref)

    out_ref[...] += contrib[None]


@jax.jit
def _run(classifications, regressions, anchors, annotations):
    B, A, C = classifications.shape
    M = annotations.shape[1]
    nblk = pl.cdiv(A, _A_BLK)
    anc_p = jnp.pad(anchors, ((0, 0), (0, 0), (0, 3)))
    ann_p = jnp.pad(annotations, ((0, 0), (0, 0), (0, 2)))
    part = pl.pallas_call(
        functools.partial(_fl_kernel, A),
        out_shape=jax.ShapeDtypeStruct((B, 1, 128), jnp.float32),
        grid=(B, nblk),
        in_specs=[
            pl.BlockSpec((1, _A_BLK, C), lambda b, i: (b, i, 0)),
            pl.BlockSpec((1, _A_BLK, 5), lambda b, i: (b, i, 0)),
            pl.BlockSpec((1, _A_BLK, 8), lambda b, i: (0, i, 0)),
            pl.BlockSpec((1, M, 8), lambda b, i: (b, 0, 0)),
        ],
        out_specs=pl.BlockSpec((1, 1, 128), lambda b, i: (b, 0, 0)),
        scratch_shapes=[
            pltpu.VMEM((16, M), jnp.float32),
            pltpu.VMEM((M, 8), jnp.float32),
        ],
        compiler_params=pltpu.CompilerParams(
            dimension_semantics=("parallel", "arbitrary"),
        ),
        name="rot_focal_loss",
    )(classifications, regressions, anc_p, ann_p)
    s = part[:, 0, :]
    cls_s, reg_s, npv = s[:, 0], s[:, 1], s[:, 2]
    cls_l = cls_s / jnp.maximum(npv, 1.0)
    reg_l = reg_s / jnp.maximum(npv * 5.0, 1.0)
    return jnp.stack([cls_l.mean(), reg_l.mean()])


def kernel(classifications, regressions, anchors, annotations):
    return _run(classifications, regressions, anchors, annotations)


# speedup vs baseline: 36.5378x; 1.0095x over previous
import functools
import math

import jax
import jax.numpy as jnp
from jax.experimental import pallas as pl
from jax.experimental.pallas import tpu as pltpu

_ALPHA = 0.25
_HOR_THR = 0.4
_ROT_THR = 0.2
_D2R = math.pi / 180.0
_A_BLK = 1024


def _fl_kernel(a_total, cls_ref, reg_ref, anc_ref, ann_ref, out_ref,
               gt_ref, rect_ref):
    b = pl.program_id(0)
    blk = pl.program_id(1)
    C = cls_ref.shape[2]
    M = ann_ref.shape[1]
    f32 = jnp.float32

    @pl.when(blk == 0)
    def _():
        annT = ann_ref[0].T
        g_cx, g_cy = annT[0:1], annT[1:2]
        g_w, g_h = annT[2:3], annT[3:4]
        g_th, g_cls = annT[4:5], annT[5:6]
        ang = g_th * _D2R
        co = jnp.cos(ang) * 0.5
        si = jnp.sin(ang) * 0.5
        p0x = g_cx - si * g_h - co * g_w
        p0y = g_cy + co * g_h - si * g_w
        p1x = g_cx + si * g_h - co * g_w
        p1y = g_cy - co * g_h - si * g_w
        p2x = 2.0 * g_cx - p0x
        p2y = 2.0 * g_cy - p0y
        p3x = 2.0 * g_cx - p1x
        p3y = 2.0 * g_cy - p1y
        bx1 = jnp.minimum(jnp.minimum(p0x, p1x), jnp.minimum(p2x, p3x))
        by1 = jnp.minimum(jnp.minimum(p0y, p1y), jnp.minimum(p2y, p3y))
        bx2 = jnp.maximum(jnp.maximum(p0x, p1x), jnp.maximum(p2x, p3x))
        by2 = jnp.maximum(jnp.maximum(p0y, p1y), jnp.maximum(p2y, p3y))
        area_b = (bx2 - bx1) * (by2 - by1)
        qarea = 0.5 * jnp.abs(
            p0x * p1y - p1x * p0y + p1x * p2y - p2x * p1y
            + p2x * p3y - p3x * p2y + p3x * p0y - p0x * p3y)
        gt_ref[...] = jnp.concatenate(
            [g_cx, g_cy, g_w, g_h, g_th, g_cls,
             p0x, p0y, p1x, p1y, p2x, p2y, p3x, p3y, qarea, qarea], axis=0)
        rect = jnp.concatenate(
            [bx1, by1, bx2, by2, area_b, area_b, area_b, area_b], axis=0)
        rect_ref[...] = rect.T

    lane = jax.lax.broadcasted_iota(jnp.int32, (1, _A_BLK), 1)
    valid = (blk * _A_BLK + lane) < a_total

    cls_t = jnp.where(valid, cls_ref[0].T, 0.5)
    reg_t = jnp.where(valid, reg_ref[0].T, 0.0)
    anc_t = jnp.where(valid, anc_ref[0].T, 0.0)

    vals16 = gt_ref[...]
    cbx1, cby1 = rect_ref[:, 0:1], rect_ref[:, 1:2]
    cbx2, cby2 = rect_ref[:, 2:3], rect_ref[:, 3:4]
    carea_b = rect_ref[:, 4:5]

    ax1, ay1 = anc_t[0:1], anc_t[1:2]
    ax2, ay2 = anc_t[2:3], anc_t[3:4]
    ath = anc_t[4:5]
    area_a = (ax2 - ax1) * (ay2 - ay1)

    iw = jnp.maximum(jnp.minimum(ax2, cbx2) - jnp.maximum(ax1, cbx1), 0.0)
    ih = jnp.maximum(jnp.minimum(ay2, cby2) - jnp.maximum(ay1, cby1), 0.0)
    inter_h = iw * ih
    ua = jnp.maximum(area_a + carea_b - inter_h, 1e-8)
    hiou = inter_h / ua
    iou_max = jnp.max(hiou, axis=0, keepdims=True)
    midx = jax.lax.broadcasted_iota(jnp.int32, (M, _A_BLK), 0)
    am = jnp.min(jnp.where(hiou == iou_max, midx, M), axis=0, keepdims=True)
    oh = (midx == am).astype(f32)
    hor_pos = (iou_max >= _HOR_THR) & valid

    gath = jnp.dot(vals16, oh, preferred_element_type=f32)
    s_cx, s_cy = gath[0:1], gath[1:2]
    s_w, s_h = gath[2:3], gath[3:4]
    s_th, s_cls = gath[4:5], gath[5:6]
    s_qa = gath[14:15]

    zero4 = jnp.zeros((4, _A_BLK), f32)
    px = jnp.concatenate([gath[6:7], gath[8:9], gath[10:11], gath[12:13]], 0)
    py = jnp.concatenate([gath[7:8], gath[9:10], gath[11:12], gath[13:14]], 0)
    px = jnp.concatenate([px, zero4], 0)
    py = jnp.concatenate([py, zero4], 0)
    n = jnp.full((1, _A_BLK), 4, jnp.int32)
    idx8 = jax.lax.broadcasted_iota(jnp.int32, (8, _A_BLK), 0)

    for use_x, bound, sign, jmax in ((True, ax1, 1.0, 5),
                                     (True, ax2, -1.0, 6),
                                     (False, ay1, 1.0, 7)):
        coord = px if use_x else py
        dp = sign * (coord - bound)
        is_last = idx8 == (n - 1)
        valid_v = idx8 < n
        qx = jnp.where(is_last, px[0:1], jnp.roll(px, -1, axis=0))
        qy = jnp.where(is_last, py[0:1], jnp.roll(py, -1, axis=0))
        dq = jnp.where(is_last, dp[0:1], jnp.roll(dp, -1, axis=0))
        in_p = dp >= 0.0
        in_q = dq >= 0.0
        denom = dp - dq
        t = dp / jnp.where(jnp.abs(denom) < 1e-9, 1e-9, denom)
        ix = px + t * (qx - px)
        iy = py + t * (qy - py)
        k0 = in_p & valid_v
        k1 = (in_p ^ in_q) & valid_v
        c0 = k0.astype(jnp.int32)
        c01 = c0 + k1.astype(jnp.int32)
        run = c01
        for s in (1, 2, 4):
            run = run + jnp.where(idx8 >= s, jnp.roll(run, s, axis=0), 0)
        posP = run - c01
        posI = posP + c0
        rows_x = []
        rows_y = []
        for j in range(jmax):
            m0 = k0 & (posP == j)
            m1 = k1 & (posI == j)
            rows_x.append(jnp.sum(jnp.where(m0, px, 0.0) + jnp.where(m1, ix, 0.0),
                                  axis=0, keepdims=True))
            rows_y.append(jnp.sum(jnp.where(m0, py, 0.0) + jnp.where(m1, iy, 0.0),
                                  axis=0, keepdims=True))
        pad = jnp.zeros((8 - jmax, _A_BLK), f32)
        px = jnp.concatenate(rows_x + [pad], axis=0)
        py = jnp.concatenate(rows_y + [pad], axis=0)
        n = jnp.minimum(jnp.sum(c01, axis=0, keepdims=True), jmax)

    dp = ay2 - py
    is_last = idx8 == (n - 1)
    valid_v = idx8 < n
    qx = jnp.where(is_last, px[0:1], jnp.roll(px, -1, axis=0))
    qy = jnp.where(is_last, py[0:1], jnp.roll(py, -1, axis=0))
    dq = jnp.where(is_last, dp[0:1], jnp.roll(dp, -1, axis=0))
    in_p = dp >= 0.0
    in_q = dq >= 0.0
    denom = dp - dq
    t = dp / jnp.where(jnp.abs(denom) < 1e-9, 1e-9, denom)
    ix = px + t * (qx - px)
    iy = py + t * (qy - py)
    k0 = in_p & valid_v
    k1 = (in_p ^ in_q) & valid_v
    idx16 = jax.lax.broadcasted_iota(jnp.int32, (16, _A_BLK), 0)
    even = (idx16 % 2) == 0
    candx = jnp.where(even, jnp.repeat(px, 2, axis=0), jnp.repeat(ix, 2, axis=0))
    candy = jnp.where(even, jnp.repeat(py, 2, axis=0), jnp.repeat(iy, 2, axis=0))
    kf = jnp.where(even, jnp.repeat(k0.astype(f32), 2, axis=0),
                   jnp.repeat(k1.astype(f32), 2, axis=0))
    wx = jnp.roll(candx, -1, axis=0)
    wy = jnp.roll(candy, -1, axis=0)
    have = jnp.roll(kf, -1, axis=0)
    for s in (1, 2, 4, 8):
        hmask = have > 0.5
        wx = jnp.where(hmask, wx, jnp.roll(wx, -s, axis=0))
        wy = jnp.where(hmask, wy, jnp.roll(wy, -s, axis=0))
        have = jnp.maximum(have, jnp.roll(have, -s, axis=0))
    cross = candx * wy - wx * candy
    inter_s = 0.5 * jnp.abs(jnp.sum(jnp.where(kf > 0.5, cross, 0.0),
                                    axis=0, keepdims=True))
    union = jnp.maximum(s_qa + area_a - inter_s, 1e-8)
    siou = inter_s / union
    pos = hor_pos & (siou >= _ROT_THR)
    npos = jnp.sum(jnp.where(pos, 1.0, 0.0))

    clsv = jnp.clip(cls_t, 1e-4, 1.0 - 1e-4)
    cio = jax.lax.broadcasted_iota(jnp.int32, (C, _A_BLK), 0)
    is_one = pos & (cio == s_cls.astype(jnp.int32))
    af = jnp.where(is_one, _ALPHA, 1.0 - _ALPHA)
    ptv = jnp.where(is_one, 1.0 - clsv, clsv)
    bce = -jnp.log(jnp.where(is_one, clsv, 1.0 - clsv))
    cls_sum = jnp.sum(jnp.where(hor_pos, af * ptv * ptv * bce, 0.0))

    aw = ax2 - ax1
    ah = ay2 - ay1
    acx = ax1 + 0.5 * aw
    acy = ay1 + 0.5 * ah
    tdx = (s_cx - acx) / aw
    tdy = (s_cy - acy) / ah
    tdw = jnp.log(jnp.maximum(s_w, 1.0) / aw)
    tdh = jnp.log(jnp.maximum(s_h, 1.0) / ah)
    tth = (s_th - ath) * _D2R
    rt = jnp.concatenate([tdy, tdx, tdh, tdw, tth], axis=0)
    diff = jnp.abs(rt - reg_t[0:5])
    sl1 = jnp.where(diff <= 1.0 / 9.0, 4.5 * diff * diff, diff - 0.5 / 9.0)
    reg_sum = jnp.sum(jnp.where(pos, sl1, 0.0))

    l1 = jax.lax.broadcasted_iota(jnp.int32, (1, 128), 1)
    contrib = (jnp.where(l1 == 0, cls_sum, 0.0)
               + jnp.where(l1 == 1, reg_sum, 0.0)
               + jnp.where(l1 == 2, npos, 0.0))

    @pl.when(blk == 0)
    def _():
        out_ref[...] = jnp.zeros_like(out_ref)

    out_ref[...] += contrib[None]


@jax.jit
def _run(classifications, regressions, anchors, annotations):
    B, A, C = classifications.shape
    M = annotations.shape[1]
    nblk = pl.cdiv(A, _A_BLK)
    anc_p = jnp.pad(anchors, ((0, 0), (0, 0), (0, 3)))
    ann_p = jnp.pad(annotations, ((0, 0), (0, 0), (0, 2)))
    part = pl.pallas_call(
        functools.partial(_fl_kernel, A),
        out_shape=jax.ShapeDtypeStruct((B, 1, 128), jnp.float32),
        grid=(B, nblk),
        in_specs=[
            pl.BlockSpec((1, _A_BLK, C), lambda b, i: (b, i, 0)),
            pl.BlockSpec((1, _A_BLK, 5), lambda b, i: (b, i, 0)),
            pl.BlockSpec((1, _A_BLK, 8), lambda b, i: (0, i, 0)),
            pl.BlockSpec((1, M, 8), lambda b, i: (b, 0, 0)),
        ],
        out_specs=pl.BlockSpec((1, 1, 128), lambda b, i: (b, 0, 0)),
        scratch_shapes=[
            pltpu.VMEM((16, M), jnp.float32),
            pltpu.VMEM((M, 8), jnp.float32),
        ],
        compiler_params=pltpu.CompilerParams(
            dimension_semantics=("arbitrary", "arbitrary"),
        ),
        name="rot_focal_loss",
    )(classifications, regressions, anc_p, ann_p)
    s = part[:, 0, :]
    cls_s, reg_s, npv = s[:, 0], s[:, 1], s[:, 2]
    cls_l = cls_s / jnp.maximum(npv, 1.0)
    reg_l = reg_s / jnp.maximum(npv * 5.0, 1.0)
    return jnp.stack([cls_l.mean(), reg_l.mean()])


def kernel(classifications, regressions, anchors, annotations):
    return _run(classifications, regressions, anchors, annotations)
